# baseline (speedup 1.0000x reference)
import jax
import jax.numpy as jnp
from jax.experimental import pallas as pl
from jax.experimental.pallas import tpu as pltpu

NUM_NODES = 512
NODE_DIM = 64
NUM_TIMES = 288
TIME_DIM = 32
WEEKDAY_DIM = 16
MODEL_DIM = 128
B, T = 32, 12
BT = B * T
ROWS_PER_STEP = 8


def _body(time_ref, wd_ref, node_ref, wn_ref, tt_ref, wt_ref, wdt_ref, ww_ref,
          out_ref, emb_s_ref, emb_t_ref):
    i = pl.program_id(0)

    @pl.when(i == 0)
    def _init():
        emb_s_ref[...] = jnp.dot(node_ref[...], wn_ref[...],
                                 preferred_element_type=jnp.float32)
        t_idx = time_ref[...]
        w_idx = wd_ref[...]
        t_iota = jax.lax.broadcasted_iota(jnp.int32, (BT, NUM_TIMES), 1)
        w_iota = jax.lax.broadcasted_iota(jnp.int32, (BT, 8), 1)
        t_oh = (t_idx == t_iota).astype(jnp.float32)
        w_oh = (w_idx == w_iota).astype(jnp.float32)
        g_t = jnp.dot(t_oh, tt_ref[...], preferred_element_type=jnp.float32)
        g_w = jnp.dot(w_oh, wdt_ref[...], preferred_element_type=jnp.float32)
        emb_t_ref[...] = (
            jnp.dot(g_t, wt_ref[...], preferred_element_type=jnp.float32)
            + jnp.dot(g_w, ww_ref[...], preferred_element_type=jnp.float32))

    rows = emb_t_ref[pl.ds(i * ROWS_PER_STEP, ROWS_PER_STEP), :]
    out_ref[...] = emb_s_ref[...][None, :, :] + rows[:, None, :]


def kernel(time, weekday, node_table, W_node, time_table, W_time,
           weekday_table, W_weekday):
    t_flat = time.reshape(BT, 1).astype(jnp.int32)
    w_flat = weekday.reshape(BT, 1).astype(jnp.int32)
    wdt_pad = jnp.pad(weekday_table, ((0, 1), (0, 0)))

    grid = (BT // ROWS_PER_STEP,)
    full = lambda shape: pl.BlockSpec(shape, lambda i: (0,) * len(shape))
    out = pl.pallas_call(
        _body,
        grid=grid,
        in_specs=[
            full((BT, 1)),
            full((BT, 1)),
            full((NUM_NODES, NODE_DIM)),
            full((NODE_DIM, MODEL_DIM)),
            full((NUM_TIMES, TIME_DIM)),
            full((TIME_DIM, MODEL_DIM)),
            full((8, WEEKDAY_DIM)),
            full((WEEKDAY_DIM, MODEL_DIM)),
        ],
        out_specs=pl.BlockSpec((ROWS_PER_STEP, NUM_NODES, MODEL_DIM),
                               lambda i: (i, 0, 0)),
        out_shape=jax.ShapeDtypeStruct((BT, NUM_NODES, MODEL_DIM),
                                       jnp.float32),
        scratch_shapes=[
            pltpu.VMEM((NUM_NODES, MODEL_DIM), jnp.float32),
            pltpu.VMEM((BT, MODEL_DIM), jnp.float32),
        ],
    )(t_flat, w_flat, node_table, W_node, time_table, W_time, wdt_pad,
      W_weekday)
    return out.reshape(B, T, NUM_NODES, MODEL_DIM)


# speedup vs baseline: 1.1446x; 1.1446x over previous
import jax
import jax.numpy as jnp
from jax.experimental import pallas as pl
from jax.experimental.pallas import tpu as pltpu

NUM_NODES = 512
NODE_DIM = 64
NUM_TIMES = 288
TIME_DIM = 32
WEEKDAY_DIM = 16
MODEL_DIM = 128
B, T = 32, 12
BT = B * T
ROWS_PER_STEP = 16


def _body(time_ref, wd_ref, node_ref, wn_ref, tt_ref, wt_ref, wdt_ref, ww_ref,
          out_ref, emb_s_ref, emb_t_ref):
    i = pl.program_id(0)

    @pl.when(i == 0)
    def _init():
        emb_s_ref[...] = jnp.dot(node_ref[...], wn_ref[...],
                                 preferred_element_type=jnp.float32)
        t_idx = time_ref[...]
        w_idx = wd_ref[...]
        t_iota = jax.lax.broadcasted_iota(jnp.int32, (BT, NUM_TIMES), 1)
        w_iota = jax.lax.broadcasted_iota(jnp.int32, (BT, 8), 1)
        t_oh = (t_idx == t_iota).astype(jnp.float32)
        w_oh = (w_idx == w_iota).astype(jnp.float32)
        g_t = jnp.dot(t_oh, tt_ref[...], preferred_element_type=jnp.float32)
        g_w = jnp.dot(w_oh, wdt_ref[...], preferred_element_type=jnp.float32)
        emb_t_ref[...] = (
            jnp.dot(g_t, wt_ref[...], preferred_element_type=jnp.float32)
            + jnp.dot(g_w, ww_ref[...], preferred_element_type=jnp.float32))

    rows = emb_t_ref[pl.ds(i * ROWS_PER_STEP, ROWS_PER_STEP), :]
    out_ref[...] = emb_s_ref[...][None, :, :] + rows[:, None, :]


def kernel(time, weekday, node_table, W_node, time_table, W_time,
           weekday_table, W_weekday):
    t_flat = time.reshape(BT, 1).astype(jnp.int32)
    w_flat = weekday.reshape(BT, 1).astype(jnp.int32)
    wdt_pad = jnp.pad(weekday_table, ((0, 1), (0, 0)))

    grid = (BT // ROWS_PER_STEP,)
    full = lambda shape: pl.BlockSpec(shape, lambda i: (0,) * len(shape))
    out = pl.pallas_call(
        _body,
        grid=grid,
        in_specs=[
            full((BT, 1)),
            full((BT, 1)),
            full((NUM_NODES, NODE_DIM)),
            full((NODE_DIM, MODEL_DIM)),
            full((NUM_TIMES, TIME_DIM)),
            full((TIME_DIM, MODEL_DIM)),
            full((8, WEEKDAY_DIM)),
            full((WEEKDAY_DIM, MODEL_DIM)),
        ],
        out_specs=pl.BlockSpec((ROWS_PER_STEP, NUM_NODES, MODEL_DIM),
                               lambda i: (i, 0, 0)),
        out_shape=jax.ShapeDtypeStruct((BT, NUM_NODES, MODEL_DIM),
                                       jnp.float32),
        scratch_shapes=[
            pltpu.VMEM((NUM_NODES, MODEL_DIM), jnp.float32),
            pltpu.VMEM((BT, MODEL_DIM), jnp.float32),
        ],
    )(t_flat, w_flat, node_table, W_node, time_table, W_time, wdt_pad,
      W_weekday)
    return out.reshape(B, T, NUM_NODES, MODEL_DIM)
